# Initial kernel scaffold; baseline (speedup 1.0000x reference)
import jax
import jax.numpy as jnp
from jax.experimental import pallas as pl


def kernel(depth_coords, encodings):
    raise NotImplementedError("write your pallas kernel here")


# speedup vs baseline: 1.7385x; 1.7385x over previous
import functools

import jax
import jax.numpy as jnp
from jax import lax
from jax.experimental import pallas as pl
from jax.experimental.pallas import tpu as pltpu
from jax.experimental.pallas import tpu_sc as plsc

NUM_CHARTS = 1024
NUM_BINS = 2048
ENC_DIM = 16
N_VERTS = 2048
LANES = 16


def _body(depth_hbm, enc_hbm, out_hbm, enc_v, depth_v, out_v):
    num_workers = 32
    charts_per_w = NUM_CHARTS // num_workers
    wid = lax.axis_index("s") * 2 + lax.axis_index("c")
    groups = N_VERTS // LANES
    iota = lax.iota(jnp.int32, LANES)
    cvecs = [jnp.full((LANES,), c, jnp.int32) for c in range(ENC_DIM)]

    def chart_body(k, carry):
        chart = wid * charts_per_w + k
        pltpu.sync_copy(enc_hbm.at[chart], enc_v)
        pltpu.sync_copy(depth_hbm.at[chart], depth_v)

        def group_body(g, c2):
            base = pl.multiple_of(g * LANES, LANES)
            d = depth_v[pl.ds(base, LANES)]
            iy = jnp.clip(d * float(NUM_BINS) - 0.5, 0.0, float(NUM_BINS - 1))
            i0 = iy.astype(jnp.int32)
            w1 = iy - i0.astype(jnp.float32)
            w0 = 1.0 - w1
            i1 = jnp.minimum(i0 + 1, NUM_BINS - 1)
            vidx = base + iota
            for c in range(ENC_DIM):
                g0 = plsc.load_gather(enc_v, [cvecs[c], i0])
                g1 = plsc.load_gather(enc_v, [cvecs[c], i1])
                plsc.store_scatter(out_v, [vidx, cvecs[c]], g0 * w0 + g1 * w1)
            return c2

        lax.fori_loop(0, groups, group_body, 0)
        pltpu.sync_copy(out_v, out_hbm.at[chart])
        return carry

    lax.fori_loop(0, charts_per_w, chart_body, 0)


@jax.jit
def kernel(depth_coords, encodings):
    mesh = plsc.VectorSubcoreMesh(core_axis_name="c", subcore_axis_name="s")
    f = pl.kernel(
        _body,
        out_type=jax.ShapeDtypeStruct((NUM_CHARTS, N_VERTS, ENC_DIM), jnp.float32),
        mesh=mesh,
        scratch_types=[
            pltpu.VMEM((ENC_DIM, NUM_BINS), jnp.float32),
            pltpu.VMEM((N_VERTS,), jnp.float32),
            pltpu.VMEM((N_VERTS, ENC_DIM), jnp.float32),
        ],
        compiler_params=pltpu.CompilerParams(
            use_tc_tiling_on_sc=False, needs_layout_passes=False
        ),
    )
    return f(depth_coords, encodings)


# speedup vs baseline: 2.1304x; 1.2254x over previous
import functools

import jax
import jax.numpy as jnp
from jax import lax
from jax.experimental import pallas as pl
from jax.experimental.pallas import tpu as pltpu
from jax.experimental.pallas import tpu_sc as plsc

NUM_CHARTS = 1024
NUM_BINS = 2048
ENC_DIM = 16
N_VERTS = 2048
LANES = 16


def _body(depth_hbm, enc_hbm, out_hbm, enc_v, depth_v, out_v):
    num_workers = 32
    charts_per_w = NUM_CHARTS // num_workers
    wid = lax.axis_index("s") * 2 + lax.axis_index("c")
    groups = N_VERTS // LANES
    iota = lax.iota(jnp.int32, LANES)
    cvecs = [jnp.full((LANES,), c, jnp.int32) for c in range(ENC_DIM)]

    def chart_body(k, carry):
        chart = wid * charts_per_w + k
        pltpu.sync_copy(enc_hbm.at[chart], enc_v)
        pltpu.sync_copy(depth_hbm.at[chart], depth_v)

        def group_body(g, c2):
            base = pl.multiple_of(g * LANES, LANES)
            d = depth_v[pl.ds(base, LANES)]
            iy = jnp.clip(d * float(NUM_BINS) - 0.5, 0.0, float(NUM_BINS - 1))
            i0 = iy.astype(jnp.int32)
            w1 = iy - i0.astype(jnp.float32)
            w0 = 1.0 - w1
            i1 = jnp.minimum(i0 + 1, NUM_BINS - 1)
            vidx = base + iota
            g0s = [plsc.load_gather(enc_v, [cvecs[c], i0]) for c in range(ENC_DIM)]
            g1s = [plsc.load_gather(enc_v, [cvecs[c], i1]) for c in range(ENC_DIM)]
            for c in range(ENC_DIM):
                plsc.store_scatter(out_v, [vidx, cvecs[c]], g0s[c] * w0 + g1s[c] * w1)
            return c2

        lax.fori_loop(0, groups, group_body, 0)
        pltpu.sync_copy(out_v, out_hbm.at[chart])
        return carry

    lax.fori_loop(0, charts_per_w, chart_body, 0)


@jax.jit
def kernel(depth_coords, encodings):
    mesh = plsc.VectorSubcoreMesh(core_axis_name="c", subcore_axis_name="s")
    f = pl.kernel(
        _body,
        out_type=jax.ShapeDtypeStruct((NUM_CHARTS, N_VERTS, ENC_DIM), jnp.float32),
        mesh=mesh,
        scratch_types=[
            pltpu.VMEM((ENC_DIM, NUM_BINS), jnp.float32),
            pltpu.VMEM((N_VERTS,), jnp.float32),
            pltpu.VMEM((N_VERTS, ENC_DIM), jnp.float32),
        ],
        compiler_params=pltpu.CompilerParams(
            use_tc_tiling_on_sc=False, needs_layout_passes=False
        ),
    )
    return f(depth_coords, encodings)


# speedup vs baseline: 2.3521x; 1.1041x over previous
import jax
import jax.numpy as jnp
from jax import lax
from jax.experimental import pallas as pl
from jax.experimental.pallas import tpu as pltpu
from jax.experimental.pallas import tpu_sc as plsc

NUM_CHARTS = 1024
NUM_BINS = 2048
ENC_DIM = 16
N_VERTS = 2048
LANES = 16
HALF = N_VERTS // 2
NUM_WORKERS = 32
CPW = NUM_CHARTS // NUM_WORKERS


def _body(depth_hbm, enc_hbm, out_hbm,
          enc0, enc1, dep0, dep1, outa, outb,
          se0, se1, sd0, sd1, so0, so1):
    encs, deps, outs = (enc0, enc1), (dep0, dep1), (outa, outb)
    sems_e, sems_d, sems_o = (se0, se1), (sd0, sd1), (so0, so1)
    wid = lax.axis_index("s") * 2 + lax.axis_index("c")
    base_chart = wid * CPW
    iota = lax.iota(jnp.int32, LANES)
    cvecs = [jnp.full((LANES,), c, jnp.int32) for c in range(ENC_DIM)]

    def start_in(k, p):
        chart = base_chart + k
        pltpu.make_async_copy(enc_hbm.at[chart], encs[p], sems_e[p]).start()
        pltpu.make_async_copy(depth_hbm.at[chart], deps[p], sems_d[p]).start()

    def wait_in(p):
        pltpu.make_async_copy(enc_hbm.at[0], encs[p], sems_e[p]).wait()
        pltpu.make_async_copy(depth_hbm.at[0], deps[p], sems_d[p]).wait()

    def wait_out(h):
        pltpu.make_async_copy(
            outs[h], out_hbm.at[0, pl.ds(0, HALF)], sems_o[h]
        ).wait()

    def compute_half(enc_v, dep_ref, out_ref, h):
        def group_body(g, c2):
            loc = pl.multiple_of(g * LANES, LANES)
            d = dep_ref[pl.ds(h * HALF + loc, LANES)]
            iy = jnp.clip(d * float(NUM_BINS) - 0.5, 0.0, float(NUM_BINS - 1))
            i0 = iy.astype(jnp.int32)
            w1 = iy - i0.astype(jnp.float32)
            w0 = 1.0 - w1
            i1 = jnp.minimum(i0 + 1, NUM_BINS - 1)
            vidx = loc + iota
            g0s = [plsc.load_gather(enc_v, [cvecs[c], i0]) for c in range(ENC_DIM)]
            g1s = [plsc.load_gather(enc_v, [cvecs[c], i1]) for c in range(ENC_DIM)]
            for c in range(ENC_DIM):
                plsc.store_scatter(out_ref, [vidx, cvecs[c]],
                                   g0s[c] * w0 + g1s[c] * w1)
            return c2

        lax.fori_loop(0, HALF // LANES, group_body, 0)

    start_in(0, 0)

    def chart_pair(j, carry):
        for p in (0, 1):
            k = j * 2 + p

            @pl.when(k + 1 < CPW)
            def _prefetch():
                start_in(k + 1, 1 - p)

            wait_in(p)
            chart = base_chart + k
            for h in (0, 1):
                @pl.when(k > 0)
                def _drain():
                    wait_out(h)

                compute_half(encs[p], deps[p], outs[h], h)
                pltpu.make_async_copy(
                    outs[h], out_hbm.at[chart, pl.ds(h * HALF, HALF)], sems_o[h]
                ).start()
        return carry

    lax.fori_loop(0, CPW // 2, chart_pair, 0)
    wait_out(0)
    wait_out(1)


@jax.jit
def kernel(depth_coords, encodings):
    mesh = plsc.VectorSubcoreMesh(core_axis_name="c", subcore_axis_name="s")
    f = pl.kernel(
        _body,
        out_type=jax.ShapeDtypeStruct((NUM_CHARTS, N_VERTS, ENC_DIM), jnp.float32),
        mesh=mesh,
        scratch_types=[
            pltpu.VMEM((ENC_DIM, NUM_BINS), jnp.float32),
            pltpu.VMEM((ENC_DIM, NUM_BINS), jnp.float32),
            pltpu.VMEM((N_VERTS,), jnp.float32),
            pltpu.VMEM((N_VERTS,), jnp.float32),
            pltpu.VMEM((HALF, ENC_DIM), jnp.float32),
            pltpu.VMEM((HALF, ENC_DIM), jnp.float32),
            pltpu.SemaphoreType.DMA,
            pltpu.SemaphoreType.DMA,
            pltpu.SemaphoreType.DMA,
            pltpu.SemaphoreType.DMA,
            pltpu.SemaphoreType.DMA,
            pltpu.SemaphoreType.DMA,
        ],
        compiler_params=pltpu.CompilerParams(
            use_tc_tiling_on_sc=False, needs_layout_passes=False
        ),
    )
    return f(depth_coords, encodings)


# speedup vs baseline: 9.8893x; 4.2044x over previous
import jax
import jax.numpy as jnp
from jax import lax
from jax.experimental import pallas as pl
from jax.experimental.pallas import tpu as pltpu
from jax.experimental.pallas import tpu_sc as plsc

NUM_CHARTS = 1024
NUM_BINS = 2048
ENC_DIM = 16
N_VERTS = 2048
LANES = 16
QUART = N_VERTS // 4
NUM_WORKERS = 32
CPW = NUM_CHARTS // NUM_WORKERS


def _body(depth_hbm, enc_hbm, out_hbm,
          enc0, enc1, dep0, dep1, outa, outb,
          se0, se1, sd0, sd1, so0, so1):
    encs, deps, outs = (enc0, enc1), (dep0, dep1), (outa, outb)
    sems_e, sems_d, sems_o = (se0, se1), (sd0, sd1), (so0, so1)
    wid = lax.axis_index("s") * 2 + lax.axis_index("c")
    base_chart = wid * CPW
    coffs = [(c // 8) * 16384 + (c % 8) * 128 for c in range(ENC_DIM)]

    def start_in(k, p):
        chart = base_chart + k
        pltpu.make_async_copy(enc_hbm.at[chart], encs[p], sems_e[p]).start()
        pltpu.make_async_copy(depth_hbm.at[chart], deps[p], sems_d[p]).start()

    def wait_in(p):
        pltpu.make_async_copy(enc_hbm.at[0], encs[p], sems_e[p]).wait()
        pltpu.make_async_copy(depth_hbm.at[0], deps[p], sems_d[p]).wait()

    def wait_out(b):
        pltpu.make_async_copy(outs[b], out_hbm.at[0, :, 0], sems_o[b]).wait()

    start_in(0, 0)

    def compute_quarter(enc_v, dep_ref, out_ref, q):
        def group_body(g, c2):
            loc = pl.multiple_of(g * LANES, LANES)
            d = dep_ref[pl.ds(q * QUART + loc, LANES)]
            iy = jnp.clip(d * float(NUM_BINS) - 0.5, 0.0, float(NUM_BINS - 1))
            i0 = iy.astype(jnp.int32)
            w1 = iy - i0.astype(jnp.float32)
            w0 = 1.0 - w1
            i1 = jnp.minimum(i0 + 1, NUM_BINS - 1)
            m0 = ((i0 >> 7) << 10) | (i0 & 127)
            m1 = ((i1 >> 7) << 10) | (i1 & 127)
            g0s = [plsc.load_gather(enc_v, [m0 + coffs[c]]) for c in range(ENC_DIM)]
            g1s = [plsc.load_gather(enc_v, [m1 + coffs[c]]) for c in range(ENC_DIM)]
            voff = ((loc >> 7) << 10) | (loc & 127)
            for c in range(ENC_DIM):
                start = pl.multiple_of(voff + (c % 8) * 128, LANES)
                out_ref[c // 8, pl.ds(start, LANES)] = g0s[c] * w0 + g1s[c] * w1
            return c2

        lax.fori_loop(0, QUART // LANES, group_body, 0)

    def chart_pair(j, carry):
        for p in (0, 1):
            k = j * 2 + p

            @pl.when(k + 1 < CPW)
            def _prefetch():
                start_in(k + 1, 1 - p)

            wait_in(p)
            chart = base_chart + k
            for q in (0, 1, 2, 3):
                b = q % 2
                if q >= 2:
                    wait_out(b)
                else:
                    @pl.when(k > 0)
                    def _drain():
                        wait_out(b)

                compute_quarter(encs[p], deps[p], outs[b], q)
                pltpu.make_async_copy(
                    outs[b], out_hbm.at[chart, :, q], sems_o[b]
                ).start()
        return carry

    lax.fori_loop(0, CPW // 2, chart_pair, 0)
    wait_out(0)
    wait_out(1)


@jax.jit
def kernel(depth_coords, encodings):
    mesh = plsc.VectorSubcoreMesh(core_axis_name="c", subcore_axis_name="s")
    f = pl.kernel(
        _body,
        out_type=jax.ShapeDtypeStruct((NUM_CHARTS, 2, 4, 4096), jnp.float32),
        mesh=mesh,
        scratch_types=[
            pltpu.VMEM((ENC_DIM * NUM_BINS,), jnp.float32),
            pltpu.VMEM((ENC_DIM * NUM_BINS,), jnp.float32),
            pltpu.VMEM((N_VERTS,), jnp.float32),
            pltpu.VMEM((N_VERTS,), jnp.float32),
            pltpu.VMEM((2, 4096), jnp.float32),
            pltpu.VMEM((2, 4096), jnp.float32),
            pltpu.SemaphoreType.DMA,
            pltpu.SemaphoreType.DMA,
            pltpu.SemaphoreType.DMA,
            pltpu.SemaphoreType.DMA,
            pltpu.SemaphoreType.DMA,
            pltpu.SemaphoreType.DMA,
        ],
        compiler_params=pltpu.CompilerParams(
            use_tc_tiling_on_sc=False, needs_layout_passes=False
        ),
    )
    enc_tiled = jnp.reshape(
        jnp.transpose(
            jnp.reshape(encodings, (NUM_CHARTS, 2, 8, 16, 128)),
            (0, 1, 3, 2, 4),
        ),
        (NUM_CHARTS, ENC_DIM * NUM_BINS),
    )
    out4 = f(depth_coords, enc_tiled)
    out_cm = jnp.reshape(
        jnp.transpose(
            jnp.reshape(out4, (NUM_CHARTS, 2, 4, 4, 8, 128)),
            (0, 1, 4, 2, 3, 5),
        ),
        (NUM_CHARTS, ENC_DIM, N_VERTS),
    )
    return jnp.transpose(out_cm, (0, 2, 1))


# speedup vs baseline: 14.5577x; 1.4721x over previous
import jax
import jax.numpy as jnp
from jax import lax
from jax.experimental import pallas as pl
from jax.experimental.pallas import tpu as pltpu
from jax.experimental.pallas import tpu_sc as plsc

NUM_CHARTS = 1024
NUM_BINS = 2048
ENC_DIM = 16
N_VERTS = 2048
LANES = 16
HALFV = N_VERTS // 2
NUM_WORKERS = 32
CPW = NUM_CHARTS // NUM_WORKERS


def _body(depth_hbm, enc_hbm, out_hbm,
          enc0, enc1, dep0, dep1, outa, outb,
          se0, se1, sd0, sd1, so0, so1):
    encs, deps, outs = (enc0, enc1), (dep0, dep1), (outa, outb)
    sems_e, sems_d, sems_o = (se0, se1), (sd0, sd1), (so0, so1)
    wid = lax.axis_index("s") * 2 + lax.axis_index("c")
    base_chart = wid * CPW
    coffs = [(c // 8) * 16384 + (c % 8) * 128 for c in range(ENC_DIM)]

    def start_in(k, p):
        chart = base_chart + k
        pltpu.make_async_copy(enc_hbm.at[chart], encs[p], sems_e[p]).start()
        pltpu.make_async_copy(
            depth_hbm.at[chart >> 3, :, chart & 7, :], deps[p], sems_d[p]
        ).start()

    def wait_in(p):
        pltpu.make_async_copy(enc_hbm.at[0], encs[p], sems_e[p]).wait()
        pltpu.make_async_copy(depth_hbm.at[0, :, 0, :], deps[p], sems_d[p]).wait()

    def wait_out(b):
        pltpu.make_async_copy(outs[b], out_hbm.at[0, :, 0], sems_o[b]).wait()

    start_in(0, 0)

    def compute_half(enc_v, dep_ref, out_ref, q):
        enc_cs = [enc_v.at[pl.ds(coffs[c], 15488)] for c in range(ENC_DIM)]

        @plsc.parallel_loop(0, HALFV // LANES, 1, unroll=2)
        def _grp(g):
            loc = pl.multiple_of(g * LANES, LANES)
            gv = q * HALFV + loc
            d = dep_ref[gv >> 7, pl.ds(pl.multiple_of(gv & 127, LANES), LANES)]
            iy = jnp.clip(d * float(NUM_BINS) - 0.5, 0.0, float(NUM_BINS - 1))
            i0 = iy.astype(jnp.int32)
            w1 = iy - i0.astype(jnp.float32)
            w0 = 1.0 - w1
            i1 = jnp.minimum(i0 + 1, NUM_BINS - 1)
            m0 = ((i0 >> 7) << 10) | (i0 & 127)
            m1 = ((i1 >> 7) << 10) | (i1 & 127)
            g0s = [plsc.load_gather(enc_cs[c], [m0]) for c in range(ENC_DIM)]
            g1s = [plsc.load_gather(enc_cs[c], [m1]) for c in range(ENC_DIM)]
            voff = ((loc >> 7) << 10) | (loc & 127)
            for c in range(ENC_DIM):
                start = pl.multiple_of(voff + (c % 8) * 128, LANES)
                out_ref[c // 8, pl.ds(start, LANES)] = g0s[c] * w0 + g1s[c] * w1

    def chart_pair(j, carry):
        for p in (0, 1):
            k = j * 2 + p

            @pl.when(k + 1 < CPW)
            def _prefetch():
                start_in(k + 1, 1 - p)

            wait_in(p)
            chart = base_chart + k
            for q in (0, 1):
                b = q
                @pl.when(k > 0)
                def _drain():
                    wait_out(b)

                compute_half(encs[p], deps[p], outs[b], q)
                pltpu.make_async_copy(
                    outs[b], out_hbm.at[chart, :, q], sems_o[b]
                ).start()
        return carry

    lax.fori_loop(0, CPW // 2, chart_pair, 0)
    wait_out(0)
    wait_out(1)


@jax.jit
def kernel(depth_coords, encodings):
    mesh = plsc.VectorSubcoreMesh(core_axis_name="c", subcore_axis_name="s")
    f = pl.kernel(
        _body,
        out_type=jax.ShapeDtypeStruct((NUM_CHARTS, 2, 2, 8192), jnp.float32),
        mesh=mesh,
        scratch_types=[
            pltpu.VMEM((ENC_DIM * NUM_BINS,), jnp.float32),
            pltpu.VMEM((ENC_DIM * NUM_BINS,), jnp.float32),
            pltpu.VMEM((16, 128), jnp.float32),
            pltpu.VMEM((16, 128), jnp.float32),
            pltpu.VMEM((2, 8192), jnp.float32),
            pltpu.VMEM((2, 8192), jnp.float32),
            pltpu.SemaphoreType.DMA,
            pltpu.SemaphoreType.DMA,
            pltpu.SemaphoreType.DMA,
            pltpu.SemaphoreType.DMA,
            pltpu.SemaphoreType.DMA,
            pltpu.SemaphoreType.DMA,
        ],
        compiler_params=pltpu.CompilerParams(
            use_tc_tiling_on_sc=False, needs_layout_passes=False
        ),
    )
    enc_tiled = jnp.reshape(
        jnp.transpose(
            jnp.reshape(encodings, (NUM_CHARTS, 2, 8, 16, 128)),
            (0, 1, 3, 2, 4),
        ),
        (NUM_CHARTS, ENC_DIM * NUM_BINS),
    )
    depth_tiled = jnp.transpose(
        jnp.reshape(depth_coords, (NUM_CHARTS // 8, 8, 16, 128)),
        (0, 2, 1, 3),
    )
    out4 = f(depth_tiled, enc_tiled)
    out_cm = jnp.reshape(
        jnp.transpose(
            jnp.reshape(out4, (NUM_CHARTS, 2, 2, 8, 8, 128)),
            (0, 1, 4, 2, 3, 5),
        ),
        (NUM_CHARTS, ENC_DIM, N_VERTS),
    )
    return jnp.transpose(out_cm, (0, 2, 1))
